# Initial kernel scaffold; baseline (speedup 1.0000x reference)
import jax
import jax.numpy as jnp
from jax.experimental import pallas as pl


def kernel(basis, outcome, table):
    raise NotImplementedError("write your pallas kernel here")


# speedup vs baseline: 3.2706x; 3.2706x over previous
import jax
import jax.numpy as jnp
from jax import lax
from jax.experimental import pallas as pl


_R = 64


def _tc_body(basis_ref, outcome_ref, table_ref, out_ref):
    r, c, _ = basis_ref.shape
    ids = basis_ref[...] * 2 + outcome_ref[...]
    ids_col = ids.reshape(r * c, 1)
    onehot = (ids_col == lax.broadcasted_iota(jnp.int32, (r * c, 8), 1))
    onehot = onehot.astype(jnp.float32)
    rows = jnp.dot(onehot, table_ref[...], preferred_element_type=jnp.float32)
    out_ref[...] = rows.reshape(r, c, 64)


def kernel(basis, outcome, table):
    n, c = basis.shape
    basis3 = basis.reshape(n, c, 1)
    outcome3 = outcome.reshape(n, c, 1)
    table_pad = jnp.zeros((8, 64), jnp.float32).at[:6].set(table)

    grid = (n // _R,)
    return pl.pallas_call(
        _tc_body,
        grid=grid,
        in_specs=[
            pl.BlockSpec((_R, c, 1), lambda i: (i, 0, 0)),
            pl.BlockSpec((_R, c, 1), lambda i: (i, 0, 0)),
            pl.BlockSpec((8, 64), lambda i: (0, 0)),
        ],
        out_specs=pl.BlockSpec((_R, c, 64), lambda i: (i, 0, 0)),
        out_shape=jax.ShapeDtypeStruct((n, c, 64), jnp.float32),
    )(basis3, outcome3, table_pad)


# speedup vs baseline: 10.5779x; 3.2342x over previous
import jax
import jax.numpy as jnp
from jax import lax
from jax.experimental import pallas as pl


_RB = 128


def _tc_body(basis_ref, outcome_ref, table_ref, out_ref):
    r, c = basis_ref.shape
    ids = basis_ref[...] * 2 + outcome_ref[...]
    tab = table_ref[...]
    tok = lax.broadcasted_iota(jnp.int32, (8, c), 0)
    for g in range(r):
        row = jnp.broadcast_to(ids[g:g + 1, :], (8, c))
        onehot = (row == tok).astype(jnp.float32)
        res = lax.dot_general(onehot, tab, (((0,), (0,)), ((), ())),
                              preferred_element_type=jnp.float32)
        out_ref[pl.ds(g * c, c), :] = res


def kernel(basis, outcome, table):
    n, c = basis.shape
    total = n * c
    basis2 = basis.reshape(total // 128, 128)
    outcome2 = outcome.reshape(total // 128, 128)
    table_pad = jnp.zeros((8, 64), jnp.float32).at[:6].set(table)

    grid = (total // 128 // _RB,)
    out = pl.pallas_call(
        _tc_body,
        grid=grid,
        in_specs=[
            pl.BlockSpec((_RB, 128), lambda i: (i, 0)),
            pl.BlockSpec((_RB, 128), lambda i: (i, 0)),
            pl.BlockSpec((8, 64), lambda i: (0, 0)),
        ],
        out_specs=pl.BlockSpec((_RB * 128, 64), lambda i: (i, 0)),
        out_shape=jax.ShapeDtypeStruct((total, 64), jnp.float32),
    )(basis2, outcome2, table_pad)
    return out.reshape(n, c, 64)
